# baseline (speedup 1.0000x reference)
import functools

import jax
import jax.numpy as jnp
from jax import lax
from jax.experimental import pallas as pl
from jax.experimental.pallas import tpu as pltpu
from jax.experimental.pallas import tpu_sc as plsc

B = 32
H = 224
W = 224
HW = H * W
CH = 7168
NCHUNK = HW // CH
VPC = CH // 16
UNROLL = 4

_mesh = plsc.VectorSubcoreMesh(core_axis_name="c", subcore_axis_name="s")


def _sc_body(xt, out, plane, xbufs, ybufs, obufs, psem, xsems, ysems, osems):
    ci = lax.axis_index("c")
    si = lax.axis_index("s")
    b = si * 2 + ci
    in_base = b * 5 * HW
    out_base = b * 3 * HW
    x_base = in_base + 3 * HW
    y_base = in_base + 4 * HW

    NG = 3 * NCHUNK

    def start_coords(g):
        p = g % 2
        k = g % NCHUNK
        cx = pltpu.async_copy(
            xt.at[pl.ds(x_base + k * CH, CH)], xbufs[p], xsems[p])
        cy = pltpu.async_copy(
            xt.at[pl.ds(y_base + k * CH, CH)], ybufs[p], ysems[p])
        return cx, cy

    plane_cp = pltpu.async_copy(xt.at[pl.ds(in_base, HW)], plane, psem)
    coord_cp = start_coords(0)
    out_cps = [None, None]
    for g in range(NG):
        c, k = divmod(g, NCHUNK)
        p = g % 2
        if k == 0:
            plane_cp.wait()
        coord_cp[0].wait()
        coord_cp[1].wait()
        if g + 1 < NG:
            coord_cp = start_coords(g + 1)
        if out_cps[p] is not None:
            out_cps[p].wait()
            out_cps[p] = None
        obuf = obufs[p]
        xbuf = xbufs[p]
        ybuf = ybufs[p]

        def vec_body(v, _, obuf=obuf, xbuf=xbuf, ybuf=ybuf):
            base = v * (16 * UNROLL)
            for u in range(UNROLL):
                o = base + u * 16
                X = xbuf[pl.ds(o, 16)]
                Y = ybuf[pl.ds(o, 16)]
                fxi = X.astype(jnp.int32)
                fyi = Y.astype(jnp.int32)
                wx = X - fxi.astype(jnp.float32)
                wy = Y - fyi.astype(jnp.float32)
                idx = fyi * W + fxi
                tl = plsc.load_gather(plane, [idx])
                tr = plsc.load_gather(plane, [idx + 1])
                bl = plsc.load_gather(plane, [idx + W])
                br = plsc.load_gather(plane, [idx + W + 1])
                top = tl + wx * (tr - tl)
                bot = bl + wx * (br - bl)
                obuf[pl.ds(o, 16)] = top + wy * (bot - top)
            return 0

        lax.fori_loop(0, VPC // UNROLL, vec_body, 0)
        if k == NCHUNK - 1 and c < 2:
            plane_cp = pltpu.async_copy(
                xt.at[pl.ds(in_base + (c + 1) * HW, HW)], plane, psem)
        out_cps[p] = pltpu.async_copy(
            obuf, out.at[pl.ds(out_base + c * HW + k * CH, CH)], osems[p])
    for cp in out_cps:
        if cp is not None:
            cp.wait()


@functools.partial(
    pl.kernel,
    out_type=jax.ShapeDtypeStruct((B * 3 * HW,), jnp.float32),
    mesh=_mesh,
    scratch_types=[
        pltpu.VMEM((HW,), jnp.float32),
        [pltpu.VMEM((CH,), jnp.float32)] * 2,
        [pltpu.VMEM((CH,), jnp.float32)] * 2,
        [pltpu.VMEM((CH,), jnp.float32)] * 2,
        pltpu.SemaphoreType.DMA,
        [pltpu.SemaphoreType.DMA] * 2,
        [pltpu.SemaphoreType.DMA] * 2,
        [pltpu.SemaphoreType.DMA] * 2,
    ],
    compiler_params=pltpu.CompilerParams(needs_layout_passes=False),
)
def _sc_bilinear(xt, out, plane, xbufs, ybufs, obufs, psem, xsems, ysems, osems):
    _sc_body(xt, out, plane, xbufs, ybufs, obufs, psem, xsems, ysems, osems)


@jax.jit
def kernel(x):
    xt = jnp.transpose(x, (0, 3, 1, 2)).reshape(-1)
    outp = _sc_bilinear(xt)
    return jnp.transpose(outp.reshape(B, 3, H, W), (0, 2, 3, 1))


# speedup vs baseline: 1.5246x; 1.5246x over previous
import functools

import jax
import jax.numpy as jnp
from jax import lax
from jax.experimental import pallas as pl
from jax.experimental.pallas import tpu as pltpu
from jax.experimental.pallas import tpu_sc as plsc

B = 32
H = 224
W = 224
HW = H * W
CH = 7168
NCHUNK = HW // CH
VPC = CH // 16
UNROLL = 4

_mesh = plsc.VectorSubcoreMesh(core_axis_name="c", subcore_axis_name="s")


def _sc_body(xt, out, plane, xbufs, ybufs, obufs, psem, xsems, ysems, osems):
    ci = lax.axis_index("c")
    si = lax.axis_index("s")
    b = si * 2 + ci
    in_base = b * 5 * HW
    out_base = b * 3 * HW
    x_base = in_base + 3 * HW
    y_base = in_base + 4 * HW

    NG = 3 * NCHUNK

    def start_coords(g):
        p = g % 2
        k = g % NCHUNK
        cx = pltpu.async_copy(
            xt.at[pl.ds(x_base + k * CH, CH)], xbufs[p], xsems[p])
        cy = pltpu.async_copy(
            xt.at[pl.ds(y_base + k * CH, CH)], ybufs[p], ysems[p])
        return cx, cy

    plane_cp = pltpu.async_copy(xt.at[pl.ds(in_base, HW)], plane, psem)
    coord_cp = start_coords(0)
    out_cps = [None, None]
    for g in range(NG):
        c, k = divmod(g, NCHUNK)
        p = g % 2
        if k == 0:
            plane_cp.wait()
        coord_cp[0].wait()
        coord_cp[1].wait()
        if g + 1 < NG:
            coord_cp = start_coords(g + 1)
        if out_cps[p] is not None:
            out_cps[p].wait()
            out_cps[p] = None
        obuf = obufs[p]
        xbuf = xbufs[p]
        ybuf = ybufs[p]

        @plsc.parallel_loop(0, CH, step=16, unroll=UNROLL)
        def vec_body(o, obuf=obuf, xbuf=xbuf, ybuf=ybuf):
            X = xbuf[pl.ds(o, 16)]
            Y = ybuf[pl.ds(o, 16)]
            fxi = X.astype(jnp.int32)
            fyi = Y.astype(jnp.int32)
            wx = X - fxi.astype(jnp.float32)
            wy = Y - fyi.astype(jnp.float32)
            idx = fyi * W + fxi
            tl = plsc.load_gather(plane, [idx])
            tr = plsc.load_gather(plane, [idx + 1])
            bl = plsc.load_gather(plane, [idx + W])
            br = plsc.load_gather(plane, [idx + W + 1])
            top = tl + wx * (tr - tl)
            bot = bl + wx * (br - bl)
            obuf[pl.ds(o, 16)] = top + wy * (bot - top)
        if k == NCHUNK - 1 and c < 2:
            plane_cp = pltpu.async_copy(
                xt.at[pl.ds(in_base + (c + 1) * HW, HW)], plane, psem)
        out_cps[p] = pltpu.async_copy(
            obuf, out.at[pl.ds(out_base + c * HW + k * CH, CH)], osems[p])
    for cp in out_cps:
        if cp is not None:
            cp.wait()


@functools.partial(
    pl.kernel,
    out_type=jax.ShapeDtypeStruct((B * 3 * HW,), jnp.float32),
    mesh=_mesh,
    scratch_types=[
        pltpu.VMEM((HW,), jnp.float32),
        [pltpu.VMEM((CH,), jnp.float32)] * 2,
        [pltpu.VMEM((CH,), jnp.float32)] * 2,
        [pltpu.VMEM((CH,), jnp.float32)] * 2,
        pltpu.SemaphoreType.DMA,
        [pltpu.SemaphoreType.DMA] * 2,
        [pltpu.SemaphoreType.DMA] * 2,
        [pltpu.SemaphoreType.DMA] * 2,
    ],
    compiler_params=pltpu.CompilerParams(needs_layout_passes=False),
)
def _sc_bilinear(xt, out, plane, xbufs, ybufs, obufs, psem, xsems, ysems, osems):
    _sc_body(xt, out, plane, xbufs, ybufs, obufs, psem, xsems, ysems, osems)


@jax.jit
def kernel(x):
    xt = jnp.transpose(x, (0, 3, 1, 2)).reshape(-1)
    outp = _sc_bilinear(xt)
    return jnp.transpose(outp.reshape(B, 3, H, W), (0, 2, 3, 1))


# speedup vs baseline: 1.5249x; 1.0002x over previous
import functools

import jax
import jax.numpy as jnp
from jax import lax
from jax.experimental import pallas as pl
from jax.experimental.pallas import tpu as pltpu
from jax.experimental.pallas import tpu_sc as plsc

B = 32
H = 224
W = 224
HW = H * W
CH = 7168
NCHUNK = HW // CH
VPC = CH // 16
UNROLL = 8

_mesh = plsc.VectorSubcoreMesh(core_axis_name="c", subcore_axis_name="s")


def _sc_body(xt, out, plane, xbufs, ybufs, obufs, psem, xsems, ysems, osems):
    ci = lax.axis_index("c")
    si = lax.axis_index("s")
    b = si * 2 + ci
    in_base = b * 5 * HW
    out_base = b * 3 * HW
    x_base = in_base + 3 * HW
    y_base = in_base + 4 * HW

    NG = 3 * NCHUNK

    def start_coords(g):
        p = g % 2
        k = g % NCHUNK
        cx = pltpu.async_copy(
            xt.at[pl.ds(x_base + k * CH, CH)], xbufs[p], xsems[p])
        cy = pltpu.async_copy(
            xt.at[pl.ds(y_base + k * CH, CH)], ybufs[p], ysems[p])
        return cx, cy

    plane_cp = pltpu.async_copy(xt.at[pl.ds(in_base, HW)], plane, psem)
    coord_cp = start_coords(0)
    out_cps = [None, None]
    for g in range(NG):
        c, k = divmod(g, NCHUNK)
        p = g % 2
        if k == 0:
            plane_cp.wait()
        coord_cp[0].wait()
        coord_cp[1].wait()
        if g + 1 < NG:
            coord_cp = start_coords(g + 1)
        if out_cps[p] is not None:
            out_cps[p].wait()
            out_cps[p] = None
        obuf = obufs[p]
        xbuf = xbufs[p]
        ybuf = ybufs[p]

        @plsc.parallel_loop(0, CH, step=16, unroll=UNROLL)
        def vec_body(o, obuf=obuf, xbuf=xbuf, ybuf=ybuf):
            X = xbuf[pl.ds(o, 16)]
            Y = ybuf[pl.ds(o, 16)]
            fxi = X.astype(jnp.int32)
            fyi = Y.astype(jnp.int32)
            wx = X - fxi.astype(jnp.float32)
            wy = Y - fyi.astype(jnp.float32)
            idx = fyi * W + fxi
            tl = plsc.load_gather(plane, [idx])
            tr = plsc.load_gather(plane, [idx + 1])
            bl = plsc.load_gather(plane, [idx + W])
            br = plsc.load_gather(plane, [idx + W + 1])
            top = tl + wx * (tr - tl)
            bot = bl + wx * (br - bl)
            obuf[pl.ds(o, 16)] = top + wy * (bot - top)
        if k == NCHUNK - 1 and c < 2:
            plane_cp = pltpu.async_copy(
                xt.at[pl.ds(in_base + (c + 1) * HW, HW)], plane, psem)
        out_cps[p] = pltpu.async_copy(
            obuf, out.at[pl.ds(out_base + c * HW + k * CH, CH)], osems[p])
    for cp in out_cps:
        if cp is not None:
            cp.wait()


@functools.partial(
    pl.kernel,
    out_type=jax.ShapeDtypeStruct((B * 3 * HW,), jnp.float32),
    mesh=_mesh,
    scratch_types=[
        pltpu.VMEM((HW,), jnp.float32),
        [pltpu.VMEM((CH,), jnp.float32)] * 2,
        [pltpu.VMEM((CH,), jnp.float32)] * 2,
        [pltpu.VMEM((CH,), jnp.float32)] * 2,
        pltpu.SemaphoreType.DMA,
        [pltpu.SemaphoreType.DMA] * 2,
        [pltpu.SemaphoreType.DMA] * 2,
        [pltpu.SemaphoreType.DMA] * 2,
    ],
    compiler_params=pltpu.CompilerParams(needs_layout_passes=False),
)
def _sc_bilinear(xt, out, plane, xbufs, ybufs, obufs, psem, xsems, ysems, osems):
    _sc_body(xt, out, plane, xbufs, ybufs, obufs, psem, xsems, ysems, osems)


@jax.jit
def kernel(x):
    xt = jnp.transpose(x, (0, 3, 1, 2)).reshape(-1)
    outp = _sc_bilinear(xt)
    return jnp.transpose(outp.reshape(B, 3, H, W), (0, 2, 3, 1))
